# baseline (speedup 1.0000x reference)
import functools

import jax
import jax.numpy as jnp
from jax import lax
from jax.experimental import pallas as pl
from jax.experimental.pallas import tpu as pltpu
from jax.experimental.pallas import tpu_sc as plsc

N_NODES = 10000
N_EDGES = 320000
C_IN = 128
NUM_GRAPHS = 512

NPAD = 10240
NC, NS = 2, 16
NW = NC * NS
LPB = 128
KPB = 80
EPAD = NW * KPB * LPB
DUMMY = N_NODES
RPS = NPAD // NS


def _sc_mesh():
    return plsc.VectorSubcoreMesh(core_axis_name="c", subcore_axis_name="s")


def _sc_degree(dst2):
    @functools.partial(
        pl.kernel,
        mesh=_sc_mesh(),
        out_type=jax.ShapeDtypeStruct((2 * NPAD, 128), jnp.float32),
        scratch_types=[
            pltpu.VMEM_SHARED((NPAD, 128), jnp.float32),
            pltpu.VMEM((KPB, LPB), jnp.int32),
            pltpu.VMEM((LPB, 128), jnp.float32),
            pltpu.VMEM((16, 128), jnp.float32),
        ],
    )
    def k(dst_hbm, out_hbm, acc, idxb, ones, zb):
        cid = lax.axis_index("c")
        sid = lax.axis_index("s")
        wid = sid * NC + cid

        @pl.loop(0, 16)
        def _(r):
            @pl.loop(0, 8)
            def _(c):
                zb[r, pl.ds(c * 16, 16)] = jnp.zeros((16,), jnp.float32)

        @pl.loop(0, LPB)
        def _(r):
            @pl.loop(0, 8)
            def _(c):
                ones[r, pl.ds(c * 16, 16)] = jnp.ones((16,), jnp.float32)

        @pl.loop(0, RPS // 16)
        def _(kk):
            pltpu.sync_copy(zb, acc.at[pl.ds(sid * RPS + kk * 16, 16)])

        pltpu.sync_copy(dst_hbm.at[pl.ds(wid * KPB, KPB)], idxb)
        plsc.subcore_barrier()

        @pl.loop(0, KPB)
        def _(j):
            pltpu.sync_copy(ones, acc.at[idxb.at[j]], add=True)

        plsc.subcore_barrier()
        pltpu.sync_copy(
            acc.at[pl.ds(sid * RPS, RPS)],
            out_hbm.at[pl.ds(cid * NPAD + sid * RPS, RPS)],
        )

    return k(dst2)


def _sc_aggregate(g2, src_list, dst2, S):
    RB = 40
    B_SMALL = 80
    SMALL_CID = 1

    @functools.partial(
        pl.kernel,
        mesh=_sc_mesh(),
        out_type=jax.ShapeDtypeStruct((2 * S * NPAD, 128), jnp.float32),
        scratch_types=[
            pltpu.VMEM_SHARED((NPAD, 128), jnp.float32),
            pltpu.VMEM((RB, LPB), jnp.int32),
            pltpu.VMEM((RB, LPB), jnp.int32),
            pltpu.VMEM((LPB, 128), jnp.float32),
            pltpu.VMEM((LPB, 128), jnp.float32),
            pltpu.VMEM((16, 128), jnp.float32),
            pltpu.SemaphoreType.DMA,
            pltpu.SemaphoreType.DMA,
        ],
    )
    def k(*refs):
        g_hbm = refs[0]
        src_hbms = refs[1:1 + S]
        dst_hbm = refs[1 + S]
        out_hbm = refs[2 + S]
        acc, sidx, didx, bufa, bufb, zb, sema, semb = refs[3 + S:]
        cid = lax.axis_index("c")
        sid = lax.axis_index("s")

        @pl.loop(0, 16)
        def _(r):
            @pl.loop(0, 8)
            def _(c):
                zb[r, pl.ds(c * 16, 16)] = jnp.zeros((16,), jnp.float32)

        def run_refill(src_hbm, base):
            pltpu.sync_copy(src_hbm.at[pl.ds(base, RB)], sidx)
            pltpu.sync_copy(dst_hbm.at[pl.ds(base, RB)], didx)

            pltpu.async_copy(g_hbm.at[sidx.at[0]], bufa, sema)
            pltpu.async_copy(g_hbm.at[sidx.at[1]], bufb, semb)

            @pl.loop(0, RB, step=2)
            def _(j):
                pltpu.make_async_copy(
                    g_hbm.at[sidx.at[j]], bufa, sema).wait()
                pltpu.sync_copy(bufa, acc.at[didx.at[j]], add=True)

                @pl.when(j + 2 < RB)
                def _():
                    pltpu.async_copy(g_hbm.at[sidx.at[j + 2]], bufa, sema)

                pltpu.make_async_copy(
                    g_hbm.at[sidx.at[j + 1]], bufb, semb).wait()
                pltpu.sync_copy(bufb, acc.at[didx.at[j + 1]], add=True)

                @pl.when(j + 3 < RB)
                def _():
                    pltpu.async_copy(g_hbm.at[sidx.at[j + 3]], bufb, semb)

        for s in range(S):
            @pl.loop(0, RPS // 16)
            def _(kk):
                pltpu.sync_copy(zb, acc.at[pl.ds(sid * RPS + kk * 16, 16)])

            plsc.subcore_barrier()

            pair_base = sid * 2 * KPB

            @pl.when(cid == SMALL_CID)
            def _():
                for r in range(B_SMALL // RB):
                    run_refill(src_hbms[s], pair_base + r * RB)

            @pl.when(cid != SMALL_CID)
            def _():
                for r in range((2 * KPB - B_SMALL) // RB):
                    run_refill(src_hbms[s], pair_base + B_SMALL + r * RB)

            plsc.subcore_barrier()
            pltpu.sync_copy(
                acc.at[pl.ds(sid * RPS, RPS)],
                out_hbm.at[pl.ds((cid * S + s) * NPAD + sid * RPS, RPS)],
            )
            plsc.subcore_barrier()

    return k(g2, *src_list, dst2)


def _tc_matmul(xin, W):
    K = xin.shape[1]
    N = W.shape[1]
    BM = 512

    def body(x_ref, w_ref, h_ref):
        h_ref[...] = jnp.dot(
            x_ref[...], w_ref[...], preferred_element_type=jnp.float32)

    return pl.pallas_call(
        body,
        grid=(NPAD // BM, N // 128),
        in_specs=[
            pl.BlockSpec((BM, K), lambda i, j: (i, 0)),
            pl.BlockSpec((K, 128), lambda i, j: (0, j)),
        ],
        out_specs=pl.BlockSpec((BM, 128), lambda i, j: (i, j)),
        out_shape=jax.ShapeDtypeStruct((NPAD, N), jnp.float32),
    )(xin, W)


def _tc_scale(h, degp):
    N = h.shape[1]
    S = N // 128
    BM = 512

    def body(h_ref, p0_ref, p1_ref, g_ref):
        deg = p0_ref[:, 0:1] + p1_ref[:, 0:1] + 1.0
        g_ref[0] = h_ref[...] * lax.rsqrt(deg)

    g = pl.pallas_call(
        body,
        grid=(NPAD // BM, S),
        in_specs=[
            pl.BlockSpec((BM, 128), lambda i, j: (i, j)),
            pl.BlockSpec((BM, 128), lambda i, j: (i, 0)),
            pl.BlockSpec((BM, 128), lambda i, j: (i + NPAD // BM, 0)),
        ],
        out_specs=pl.BlockSpec((1, BM, 128), lambda i, j: (j, i, 0)),
        out_shape=jax.ShapeDtypeStruct((S, NPAD, 128), jnp.float32),
    )(h, degp, degp)
    return g.reshape(S * NPAD, 128)


def _tc_matmul_scale(xin, W, degp):
    K = xin.shape[1]
    N = W.shape[1]
    S = N // 128
    BM = 512

    def body(x_ref, w_ref, p0_ref, p1_ref, h_ref, g_ref):
        h = jnp.dot(x_ref[...], w_ref[...], preferred_element_type=jnp.float32)
        deg = p0_ref[:, 0:1] + p1_ref[:, 0:1] + 1.0
        dis = lax.rsqrt(deg)
        h_ref[...] = h
        g_ref[0] = h * dis

    h, g = pl.pallas_call(
        body,
        grid=(NPAD // BM, S),
        in_specs=[
            pl.BlockSpec((BM, K), lambda i, j: (i, 0)),
            pl.BlockSpec((K, 128), lambda i, j: (0, j)),
            pl.BlockSpec((BM, 128), lambda i, j: (i, 0)),
            pl.BlockSpec((BM, 128), lambda i, j: (i + NPAD // BM, 0)),
        ],
        out_specs=[
            pl.BlockSpec((BM, 128), lambda i, j: (i, j)),
            pl.BlockSpec((1, BM, 128), lambda i, j: (j, i, 0)),
        ],
        out_shape=[
            jax.ShapeDtypeStruct((NPAD, N), jnp.float32),
            jax.ShapeDtypeStruct((S, NPAD, 128), jnp.float32),
        ],
    )(xin, W, degp, degp)
    return h, g.reshape(S * NPAD, 128)


def _tc_combine(agg, h, degp, b, prelu_a2, S):
    BM = 512
    NB = NPAD // BM

    def body(a0_ref, a1_ref, h_ref, p0_ref, p1_ref, b_ref, a_ref, o_ref):
        deg = p0_ref[:, 0:1] + p1_ref[:, 0:1] + 1.0
        dis = lax.rsqrt(deg)
        v = (a0_ref[0] + a1_ref[0]) * dis + h_ref[...] * (1.0 / deg)
        v = v + b_ref[...]
        a = a_ref[0, 0]
        o_ref[...] = jnp.where(v >= 0, v, a * v)

    agg3 = agg.reshape(2 * S, NPAD, 128)
    return pl.pallas_call(
        body,
        grid=(NB, S),
        in_specs=[
            pl.BlockSpec((1, BM, 128), lambda i, j: (j, i, 0)),
            pl.BlockSpec((1, BM, 128), lambda i, j: (S + j, i, 0)),
            pl.BlockSpec((BM, 128), lambda i, j: (i, j)),
            pl.BlockSpec((BM, 128), lambda i, j: (i, 0)),
            pl.BlockSpec((BM, 128), lambda i, j: (i + NB, 0)),
            pl.BlockSpec((128,), lambda i, j: (j,)),
            pl.BlockSpec((1, 1), lambda i, j: (0, 0)),
        ],
        out_specs=pl.BlockSpec((BM, 128), lambda i, j: (i, j)),
        out_shape=jax.ShapeDtypeStruct((NPAD, S * 128), jnp.float32),
    )(agg3, agg3, h, degp, degp, b, prelu_a2)


def _tc_seg_bounds(batch):
    CH = 1024
    NCH = NPAD // CH

    def body(b_ref, lo_ref, hi_ref):
        seg = lax.broadcasted_iota(jnp.int32, (1, NUM_GRAPHS), 1)
        lo = jnp.zeros((1, NUM_GRAPHS), jnp.int32)
        hi = jnp.zeros((1, NUM_GRAPHS), jnp.int32)

        def step(c, carry):
            lo, hi = carry
            b = b_ref[pl.ds(c * CH, CH)][:, None]
            lo = lo + jnp.sum((b < seg).astype(jnp.int32), axis=0,
                              keepdims=True)
            hi = hi + jnp.sum((b <= seg).astype(jnp.int32), axis=0,
                              keepdims=True)
            return lo, hi

        lo, hi = lax.fori_loop(0, NCH, step, (lo, hi))
        lo_ref[...] = lo
        hi_ref[...] = hi

    return pl.pallas_call(
        body,
        out_shape=[jax.ShapeDtypeStruct((1, NUM_GRAPHS), jnp.int32),
                   jax.ShapeDtypeStruct((1, NUM_GRAPHS), jnp.int32)],
    )(batch)


def _tc_segmax(h, batch):
    batchp = jnp.concatenate(
        [batch, jnp.full((NPAD - N_NODES,), NUM_GRAPHS, jnp.int32)])
    lo, hi = _tc_seg_bounds(batchp)

    SEGB = 8

    def body(lo_ref, hi_ref, x_ref, o_ref):
        g = pl.program_id(0)
        rows = lax.broadcasted_iota(jnp.int32, (8, 512), 0)

        for r in range(SEGB):
            s = g * SEGB + r
            start = lo_ref[0, s]
            end = hi_ref[0, s]
            base0 = (start // 8) * 8

            def step(k, acc):
                base = pl.multiple_of(base0 + k * 8, 8)
                chunk = x_ref[pl.ds(base, 8), :]
                ridx = rows + base
                m = (ridx >= start) & (ridx < end)
                return jnp.maximum(acc, jnp.where(m, chunk, -jnp.inf))

            acc = jnp.full((8, 512), -jnp.inf, jnp.float32)
            nch = (end - base0 + 7) // 8
            acc = lax.fori_loop(0, nch, step, acc)
            o_ref[pl.ds(r, 1), :] = jnp.max(acc, axis=0, keepdims=True)

    grid_spec = pltpu.PrefetchScalarGridSpec(
        num_scalar_prefetch=2,
        grid=(NUM_GRAPHS // SEGB,),
        in_specs=[pl.BlockSpec((NPAD, 512), lambda g, lo, hi: (0, 0))],
        out_specs=pl.BlockSpec((SEGB, 512), lambda g, lo, hi: (g, 0)),
    )
    return pl.pallas_call(
        body,
        grid_spec=grid_spec,
        out_shape=jax.ShapeDtypeStruct((NUM_GRAPHS, 512), jnp.float32),
    )(lo, hi, h)


def kernel(x, edge_index, batch, W1, b1, W2, b2, W3, b3, prelu_a):
    xp = jnp.concatenate(
        [x, jnp.zeros((NPAD - N_NODES, C_IN), jnp.float32)], axis=0)
    pad = EPAD - N_EDGES
    src = jnp.concatenate(
        [edge_index[0], jnp.zeros((pad,), jnp.int32)]).reshape(NW * KPB, LPB)
    dst = jnp.concatenate(
        [edge_index[1], jnp.full((pad,), DUMMY, jnp.int32)]).reshape(
            NW * KPB, LPB)

    hw1 = _tc_matmul(xp, W1)
    degp = _sc_degree(dst)
    prelu_a2 = prelu_a.reshape(1, 1)

    h = xp
    for li, (W, b) in enumerate(((W1, b1), (W2, b2), (W3, b3))):
        S = W.shape[1] // 128
        if li == 0:
            hw, g2 = hw1, _tc_scale(hw1, degp)
        else:
            hw, g2 = _tc_matmul_scale(h, W, degp)
        src_list = [src + s * NPAD for s in range(S)]
        agg = _sc_aggregate(g2, src_list, dst, S)
        h = _tc_combine(agg, hw, degp, b, prelu_a2, S)

    return _tc_segmax(h, batch)


# speedup vs baseline: 1.0951x; 1.0951x over previous
import functools

import jax
import jax.numpy as jnp
from jax import lax
from jax.experimental import pallas as pl
from jax.experimental.pallas import tpu as pltpu
from jax.experimental.pallas import tpu_sc as plsc

N_NODES = 10000
N_EDGES = 320000
C_IN = 128
NUM_GRAPHS = 512

NPAD = 10240
NC, NS = 2, 16
NW = NC * NS
LPB = 128
KPB = 80
EPAD = NW * KPB * LPB
DUMMY = N_NODES
RPS = NPAD // NS


def _sc_mesh():
    return plsc.VectorSubcoreMesh(core_axis_name="c", subcore_axis_name="s")


def _sc_degree(dst2):
    @functools.partial(
        pl.kernel,
        mesh=_sc_mesh(),
        out_type=jax.ShapeDtypeStruct((2 * NPAD, 128), jnp.float32),
        scratch_types=[
            pltpu.VMEM_SHARED((NPAD, 128), jnp.float32),
            pltpu.VMEM((KPB, LPB), jnp.int32),
            pltpu.VMEM((LPB, 128), jnp.float32),
            pltpu.VMEM((16, 128), jnp.float32),
        ],
    )
    def k(dst_hbm, out_hbm, acc, idxb, ones, zb):
        cid = lax.axis_index("c")
        sid = lax.axis_index("s")
        wid = sid * NC + cid

        @pl.loop(0, 16)
        def _(r):
            @pl.loop(0, 8)
            def _(c):
                zb[r, pl.ds(c * 16, 16)] = jnp.zeros((16,), jnp.float32)

        @pl.loop(0, LPB)
        def _(r):
            @pl.loop(0, 8)
            def _(c):
                ones[r, pl.ds(c * 16, 16)] = jnp.ones((16,), jnp.float32)

        @pl.loop(0, RPS // 16)
        def _(kk):
            pltpu.sync_copy(zb, acc.at[pl.ds(sid * RPS + kk * 16, 16)])

        pltpu.sync_copy(dst_hbm.at[pl.ds(wid * KPB, KPB)], idxb)
        plsc.subcore_barrier()

        @pl.loop(0, KPB)
        def _(j):
            pltpu.sync_copy(ones, acc.at[idxb.at[j]], add=True)

        plsc.subcore_barrier()
        pltpu.sync_copy(
            acc.at[pl.ds(sid * RPS, RPS)],
            out_hbm.at[pl.ds(cid * NPAD + sid * RPS, RPS)],
        )

    return k(dst2)


def _sc_aggregate(g2, src_list, dst2, S):
    RB = 40
    B_SMALL = 80
    SMALL_CID = 1

    @functools.partial(
        pl.kernel,
        mesh=_sc_mesh(),
        out_type=jax.ShapeDtypeStruct((2 * S * NPAD, 128), jnp.float32),
        scratch_types=[
            pltpu.VMEM_SHARED((NPAD, 128), jnp.float32),
            pltpu.VMEM((RB, LPB), jnp.int32),
            pltpu.VMEM((RB, LPB), jnp.int32),
            pltpu.VMEM((LPB, 128), jnp.float32),
            pltpu.VMEM((LPB, 128), jnp.float32),
            pltpu.VMEM((16, 128), jnp.float32),
            pltpu.SemaphoreType.DMA,
            pltpu.SemaphoreType.DMA,
        ],
    )
    def k(*refs):
        g_hbm = refs[0]
        src_hbms = refs[1:1 + S]
        dst_hbm = refs[1 + S]
        out_hbm = refs[2 + S]
        acc, sidx, didx, bufa, bufb, zb, sema, semb = refs[3 + S:]
        cid = lax.axis_index("c")
        sid = lax.axis_index("s")

        @pl.loop(0, 16)
        def _(r):
            @pl.loop(0, 8)
            def _(c):
                zb[r, pl.ds(c * 16, 16)] = jnp.zeros((16,), jnp.float32)

        def run_refill(src_hbm, base):
            pltpu.sync_copy(src_hbm.at[pl.ds(base, RB)], sidx)
            pltpu.sync_copy(dst_hbm.at[pl.ds(base, RB)], didx)

            pltpu.async_copy(g_hbm.at[sidx.at[0]], bufa, sema)
            pltpu.async_copy(g_hbm.at[sidx.at[1]], bufb, semb)

            @pl.loop(0, RB, step=2)
            def _(j):
                pltpu.make_async_copy(
                    g_hbm.at[sidx.at[j]], bufa, sema).wait()
                pltpu.sync_copy(bufa, acc.at[didx.at[j]], add=True)

                @pl.when(j + 2 < RB)
                def _():
                    pltpu.async_copy(g_hbm.at[sidx.at[j + 2]], bufa, sema)

                pltpu.make_async_copy(
                    g_hbm.at[sidx.at[j + 1]], bufb, semb).wait()
                pltpu.sync_copy(bufb, acc.at[didx.at[j + 1]], add=True)

                @pl.when(j + 3 < RB)
                def _():
                    pltpu.async_copy(g_hbm.at[sidx.at[j + 3]], bufb, semb)

        for s in range(S):
            @pl.loop(0, RPS // 16)
            def _(kk):
                pltpu.sync_copy(zb, acc.at[pl.ds(sid * RPS + kk * 16, 16)])

            plsc.subcore_barrier()

            pair_base = sid * 2 * KPB

            @pl.when(cid == SMALL_CID)
            def _():
                for r in range(B_SMALL // RB):
                    run_refill(src_hbms[s], pair_base + r * RB)

            @pl.when(cid != SMALL_CID)
            def _():
                for r in range((2 * KPB - B_SMALL) // RB):
                    run_refill(src_hbms[s], pair_base + B_SMALL + r * RB)

            plsc.subcore_barrier()
            pltpu.sync_copy(
                acc.at[pl.ds(sid * RPS, RPS)],
                out_hbm.at[pl.ds((cid * S + s) * NPAD + sid * RPS, RPS)],
            )
            plsc.subcore_barrier()

    return k(g2, *src_list, dst2)


def _tc_matmul_scale(xin, W, degp):
    K = xin.shape[1]
    N = W.shape[1]
    S = N // 128
    BM = 512

    def body(x_ref, w_ref, p0_ref, p1_ref, h_ref, g_ref):
        h = jnp.dot(x_ref[...], w_ref[...], preferred_element_type=jnp.float32)
        deg = p0_ref[:, 0:1] + p1_ref[:, 0:1] + 1.0
        dis = lax.rsqrt(deg)
        h_ref[...] = h
        g_ref[0] = h * dis

    h, g = pl.pallas_call(
        body,
        grid=(NPAD // BM, S),
        in_specs=[
            pl.BlockSpec((BM, K), lambda i, j: (i, 0)),
            pl.BlockSpec((K, 128), lambda i, j: (0, j)),
            pl.BlockSpec((BM, 128), lambda i, j: (i, 0)),
            pl.BlockSpec((BM, 128), lambda i, j: (i + NPAD // BM, 0)),
        ],
        out_specs=[
            pl.BlockSpec((BM, 128), lambda i, j: (i, j)),
            pl.BlockSpec((1, BM, 128), lambda i, j: (j, i, 0)),
        ],
        out_shape=[
            jax.ShapeDtypeStruct((NPAD, N), jnp.float32),
            jax.ShapeDtypeStruct((S, NPAD, 128), jnp.float32),
        ],
    )(xin, W, degp, degp)
    return h, g.reshape(S * NPAD, 128)


def _tc_combine(agg, h, degp, b, prelu_a2, S):
    BM = 512
    NB = NPAD // BM

    def body(a0_ref, a1_ref, h_ref, p0_ref, p1_ref, b_ref, a_ref, o_ref):
        deg = p0_ref[:, 0:1] + p1_ref[:, 0:1] + 1.0
        dis = lax.rsqrt(deg)
        v = (a0_ref[0] + a1_ref[0]) * dis + h_ref[...] * (1.0 / deg)
        v = v + b_ref[...]
        a = a_ref[0, 0]
        o_ref[...] = jnp.where(v >= 0, v, a * v)

    agg3 = agg.reshape(2 * S, NPAD, 128)
    return pl.pallas_call(
        body,
        grid=(NB, S),
        in_specs=[
            pl.BlockSpec((1, BM, 128), lambda i, j: (j, i, 0)),
            pl.BlockSpec((1, BM, 128), lambda i, j: (S + j, i, 0)),
            pl.BlockSpec((BM, 128), lambda i, j: (i, j)),
            pl.BlockSpec((BM, 128), lambda i, j: (i, 0)),
            pl.BlockSpec((BM, 128), lambda i, j: (i + NB, 0)),
            pl.BlockSpec((128,), lambda i, j: (j,)),
            pl.BlockSpec((1, 1), lambda i, j: (0, 0)),
        ],
        out_specs=pl.BlockSpec((BM, 128), lambda i, j: (i, j)),
        out_shape=jax.ShapeDtypeStruct((NPAD, S * 128), jnp.float32),
    )(agg3, agg3, h, degp, degp, b, prelu_a2)


def _tc_seg_bounds(batch):
    CH = 1024
    NCH = NPAD // CH

    def body(b_ref, lo_ref, hi_ref):
        seg = lax.broadcasted_iota(jnp.int32, (1, NUM_GRAPHS), 1)
        lo = jnp.zeros((1, NUM_GRAPHS), jnp.int32)
        hi = jnp.zeros((1, NUM_GRAPHS), jnp.int32)

        def step(c, carry):
            lo, hi = carry
            b = b_ref[pl.ds(c * CH, CH)][:, None]
            lo = lo + jnp.sum((b < seg).astype(jnp.int32), axis=0,
                              keepdims=True)
            hi = hi + jnp.sum((b <= seg).astype(jnp.int32), axis=0,
                              keepdims=True)
            return lo, hi

        lo, hi = lax.fori_loop(0, NCH, step, (lo, hi))
        lo_ref[...] = lo
        hi_ref[...] = hi

    return pl.pallas_call(
        body,
        out_shape=[jax.ShapeDtypeStruct((1, NUM_GRAPHS), jnp.int32),
                   jax.ShapeDtypeStruct((1, NUM_GRAPHS), jnp.int32)],
    )(batch)


def _tc_segmax(h, batch):
    batchp = jnp.concatenate(
        [batch, jnp.full((NPAD - N_NODES,), NUM_GRAPHS, jnp.int32)])
    lo, hi = _tc_seg_bounds(batchp)

    SEGB = 8

    def body(lo_ref, hi_ref, x_ref, o_ref):
        g = pl.program_id(0)
        rows = lax.broadcasted_iota(jnp.int32, (8, 512), 0)

        for r in range(SEGB):
            s = g * SEGB + r
            start = lo_ref[0, s]
            end = hi_ref[0, s]
            base0 = (start // 8) * 8

            def step(k, acc):
                base = pl.multiple_of(base0 + k * 8, 8)
                chunk = x_ref[pl.ds(base, 8), :]
                ridx = rows + base
                m = (ridx >= start) & (ridx < end)
                return jnp.maximum(acc, jnp.where(m, chunk, -jnp.inf))

            acc = jnp.full((8, 512), -jnp.inf, jnp.float32)
            nch = (end - base0 + 7) // 8
            acc = lax.fori_loop(0, nch, step, acc)
            o_ref[pl.ds(r, 1), :] = jnp.max(acc, axis=0, keepdims=True)

    grid_spec = pltpu.PrefetchScalarGridSpec(
        num_scalar_prefetch=2,
        grid=(NUM_GRAPHS // SEGB,),
        in_specs=[pl.BlockSpec((NPAD, 512), lambda g, lo, hi: (0, 0))],
        out_specs=pl.BlockSpec((SEGB, 512), lambda g, lo, hi: (g, 0)),
    )
    return pl.pallas_call(
        body,
        grid_spec=grid_spec,
        out_shape=jax.ShapeDtypeStruct((NUM_GRAPHS, 512), jnp.float32),
    )(lo, hi, h)


def kernel(x, edge_index, batch, W1, b1, W2, b2, W3, b3, prelu_a):
    xp = jnp.concatenate(
        [x, jnp.zeros((NPAD - N_NODES, C_IN), jnp.float32)], axis=0)
    pad = EPAD - N_EDGES
    src = jnp.concatenate(
        [edge_index[0], jnp.zeros((pad,), jnp.int32)]).reshape(NW * KPB, LPB)
    dst = jnp.concatenate(
        [edge_index[1], jnp.full((pad,), DUMMY, jnp.int32)]).reshape(
            NW * KPB, LPB)

    degp = _sc_degree(dst)
    prelu_a2 = prelu_a.reshape(1, 1)

    h = xp
    for W, b in ((W1, b1), (W2, b2), (W3, b3)):
        S = W.shape[1] // 128
        hw, g2 = _tc_matmul_scale(h, W, degp)
        src_list = [src + s * NPAD for s in range(S)]
        agg = _sc_aggregate(g2, src_list, dst, S)
        h = _tc_combine(agg, hw, degp, b, prelu_a2, S)

    return _tc_segmax(h, batch)
